# baseline (speedup 1.0000x reference)
import functools

import jax
import jax.numpy as jnp
from jax import lax
from jax.experimental import pallas as pl
from jax.experimental.pallas import tpu as pltpu
from jax.experimental.pallas import tpu_sc as plsc

_B = 16384
_CD = 64
_WIDE = 2 * _CD
_NC = 1000000
_NW = 32
_ROWS = 2 * _B
_RPW = _ROWS // _NW
_PHASE = 256
_NPHASE = _RPW // _PHASE
_PRED_PAD = 128

_LO = 499712
_HI = 999936
_TCL = 31232
_TCGRID = _LO // _TCL
_SCCOL0 = _LO // 128
_SCCOLS = (_HI - _LO) // 128
_SCJ = (_SCCOLS + _NW - 1) // _NW


def _tc_transpose_main(tt):
    def body(x_ref, o_ref):
        o_ref[...] = jnp.transpose(x_ref[...])

    return pl.pallas_call(
        body,
        grid=(_TCGRID,),
        in_specs=[pl.BlockSpec((_CD, _TCL), lambda i: (0, i))],
        out_specs=pl.BlockSpec((_TCL, _CD), lambda i: (i, 0)),
        out_shape=jax.ShapeDtypeStruct((_LO, _CD), jnp.float32),
    )(tt)


def _tc_transpose_tail(tt):
    def body(x_ref, o_ref):
        o_ref[...] = jnp.transpose(x_ref[...])

    return pl.pallas_call(
        body,
        grid=(1,),
        in_specs=[pl.BlockSpec((_CD, 128), lambda i: (0, _HI // 128))],
        out_specs=pl.BlockSpec((128, _CD), lambda i: (0, 0)),
        out_shape=jax.ShapeDtypeStruct((128, _CD), jnp.float32),
    )(tt)


def _sc_transpose(tt):
    mesh = plsc.VectorSubcoreMesh(core_axis_name="c", subcore_axis_name="s")

    @functools.partial(
        pl.kernel,
        mesh=mesh,
        out_type=jax.ShapeDtypeStruct(((_HI - _LO) // 2, 2, _CD), jnp.float32),
        scratch_types=[
            pltpu.VMEM((_CD, 128), jnp.float32),
            pltpu.VMEM((_CD, 2, _CD), jnp.float32),
            pltpu.SemaphoreType.DMA,
            pltpu.SemaphoreType.DMA,
        ],
        compiler_params=pltpu.CompilerParams(needs_layout_passes=False),
    )
    def k(tt_hbm, out_hbm, s_v, w_v, sem_s, sem_w):
        wid = lax.axis_index("s") * 2 + lax.axis_index("c")
        iota = lax.broadcasted_iota(jnp.int32, (16,), 0)

        def compact(w2, _):
            col = jnp.zeros((16,), jnp.int32) + 2 * w2
            for cblk in range(4):
                rows = iota + 16 * cblk
                v0 = plsc.load_gather(s_v, [rows, col])
                w_v[w2, 0, pl.ds(16 * cblk, 16)] = v0
                v1 = plsc.load_gather(s_v, [rows, col + 1])
                w_v[w2, 1, pl.ds(16 * cblk, 16)] = v1
            return 0

        def body(j, _):
            tcq = _SCCOL0 + wid + _NW * j

            @pl.when(tcq < _SCCOL0 + _SCCOLS)
            def _():
                stages = []
                for tr in range(8):
                    stages.append(
                        pltpu.async_copy(
                            tt_hbm.at[pl.ds(8 * tr, 8), pl.ds(128 * tcq, 128)],
                            s_v.at[pl.ds(8 * tr, 8), :],
                            sem_s,
                        )
                    )
                for c in stages:
                    c.wait()

                @pl.when(j > 0)
                def _():
                    pltpu.make_async_copy(
                        w_v, out_hbm.at[pl.ds(0, _CD)], sem_w
                    ).wait()

                lax.fori_loop(0, _CD, compact, 0)
                pltpu.async_copy(
                    w_v, out_hbm.at[pl.ds((tcq - _SCCOL0) * _CD, _CD)], sem_w
                )

            return 0

        lax.fori_loop(0, _SCJ, body, 0)
        pltpu.make_async_copy(w_v, out_hbm.at[pl.ds(0, _CD)], sem_w).wait()

    return k(tt)


def _sc_gather(t0, t1w, t2, idx2):
    mesh = plsc.VectorSubcoreMesh(core_axis_name="c", subcore_axis_name="s")

    @functools.partial(
        pl.kernel,
        mesh=mesh,
        out_type=jax.ShapeDtypeStruct((_ROWS, 2, _CD), jnp.float32),
        scratch_types=[
            pltpu.VMEM((_RPW,), jnp.int32),
            pltpu.VMEM((_PHASE, 2, _CD), jnp.float32),
            pltpu.SemaphoreType.DMA,
        ],
        compiler_params=pltpu.CompilerParams(needs_layout_passes=False),
    )
    def k(t0_hbm, t1_hbm, t2_hbm, idx_hbm, out_hbm, idx_v, rows_v, sem):
        wid = lax.axis_index("s") * 2 + lax.axis_index("c")
        pltpu.sync_copy(idx_hbm.at[wid], idx_v)
        lanes = lax.broadcasted_iota(jnp.int32, (16,), 0)
        zeros = jnp.zeros((16,), jnp.int32)
        ngroup = _PHASE // 16
        lag = 4

        def drain(i, _):
            pltpu.make_async_copy(t1_hbm.at[0], rows_v.at[0], sem).wait()
            return 0

        for ph in range(_NPHASE):
            def group(g, _, base=ph * _PHASE):
                v = idx_v[pl.ds(base + g * 16, 16)]
                for l in range(16):
                    r = jnp.sum(jnp.where(lanes == l, v, zeros))
                    slot = g * 16 + l

                    @pl.when(r < _LO)
                    def _():
                        pltpu.async_copy(t0_hbm.at[r], rows_v.at[slot, 0], sem)
                        pltpu.async_copy(t0_hbm.at[r], rows_v.at[slot, 1], sem)

                    @pl.when((r >= _LO) & (r < _HI))
                    def _():
                        pltpu.async_copy(
                            t1_hbm.at[(r - _LO) >> 1], rows_v.at[slot], sem
                        )

                    @pl.when(r >= _HI)
                    def _():
                        pltpu.async_copy(t2_hbm.at[r - _HI], rows_v.at[slot, 0], sem)
                        pltpu.async_copy(t2_hbm.at[r - _HI], rows_v.at[slot, 1], sem)

                @pl.when(g >= lag)
                def _():
                    lax.fori_loop(0, 16, drain, 0)

                return 0

            lax.fori_loop(0, ngroup, group, 0)
            lax.fori_loop(0, lag * 16, drain, 0)
            pltpu.sync_copy(
                rows_v, out_hbm.at[pl.ds(wid * _RPW + ph * _PHASE, _PHASE)]
            )

    return k(t0, t1w, t2, idx2)


def _tc_score(g0w, g1w, p0, p1, pred_idx2, pred_pad, w_p, w0, w1, bias):
    bb = 2048
    grid = _B // bb

    def body(g0_ref, g1_ref, p0_ref, p1_ref, pi_ref, pt_ref, wp_ref, w0_ref,
             w1_ref, b_ref, o_ref):
        p = jnp.dot(pt_ref[...], wp_ref[...], preferred_element_type=jnp.float32)
        sel0 = jnp.where(p0_ref[...] == 0, g0_ref[:, :_CD], g0_ref[:, _CD:])
        sel1 = jnp.where(p1_ref[...] == 0, g1_ref[:, :_CD], g1_ref[:, _CD:])
        onehot = (
            pi_ref[...] == lax.broadcasted_iota(jnp.int32, (bb, _PRED_PAD), 1)
        ).astype(jnp.float32)
        acc = (
            jnp.dot(sel0, w0_ref[...], preferred_element_type=jnp.float32)
            + jnp.dot(sel1, w1_ref[...], preferred_element_type=jnp.float32)
            + jnp.dot(onehot, p, preferred_element_type=jnp.float32)
            + b_ref[...]
        )
        o_ref[...] = jnp.tanh(acc)

    return pl.pallas_call(
        body,
        grid=(grid,),
        in_specs=[
            pl.BlockSpec((bb, _WIDE), lambda i: (i, 0)),
            pl.BlockSpec((bb, _WIDE), lambda i: (i, 0)),
            pl.BlockSpec((bb, 1), lambda i: (i, 0)),
            pl.BlockSpec((bb, 1), lambda i: (i, 0)),
            pl.BlockSpec((bb, 1), lambda i: (i, 0)),
            pl.BlockSpec((_PRED_PAD, _CD), lambda i: (0, 0)),
            pl.BlockSpec((_CD, _CD), lambda i: (0, 0)),
            pl.BlockSpec((_CD, _CD), lambda i: (0, 0)),
            pl.BlockSpec((_CD, _CD), lambda i: (0, 0)),
            pl.BlockSpec((1, _CD), lambda i: (0, 0)),
        ],
        out_specs=pl.BlockSpec((bb, _CD), lambda i: (i, 0)),
        out_shape=jax.ShapeDtypeStruct((_B, _CD), jnp.float32),
    )(g0w, g1w, p0, p1, pred_idx2, pred_pad, w_p, w0, w1, bias)


def kernel(triplet_idx, predicate_idx, constant_table, predicate_table, W, b):
    ti = triplet_idx.astype(jnp.int32)
    idx_all = jnp.concatenate([ti[:, 0], ti[:, 1]])
    idx2 = idx_all.reshape(_NW, _RPW)
    tt = constant_table.T
    t0 = _tc_transpose_main(tt)
    t2 = _tc_transpose_tail(tt)
    t1w = _sc_transpose(tt)
    g3 = _sc_gather(t0, t1w, t2, idx2)
    g = g3.reshape(_ROWS, _WIDE)
    g0w = g[:_B]
    g1w = g[_B:]
    p0 = (ti[:, 0] & 1).reshape(_B, 1)
    p1 = (ti[:, 1] & 1).reshape(_B, 1)
    pred_pad = jnp.zeros((_PRED_PAD, _CD), jnp.float32).at[
        : predicate_table.shape[0]
    ].set(predicate_table)
    pi2 = predicate_idx.astype(jnp.int32).reshape(_B, 1)
    return _tc_score(
        g0w, g1w, p0, p1, pi2, pred_pad,
        W[:_CD], W[_CD : 2 * _CD], W[2 * _CD :], b.reshape(1, _CD),
    )


# speedup vs baseline: 3.5080x; 3.5080x over previous
import functools

import jax
import jax.numpy as jnp
from jax import lax
from jax.experimental import pallas as pl
from jax.experimental.pallas import tpu as pltpu
from jax.experimental.pallas import tpu_sc as plsc

_B = 16384
_CD = 64
_WIDE = 2 * _CD
_NC = 1000000
_NW = 32
_ROWS = 2 * _B
_RPW = _ROWS // _NW
_PHASE = 512
_NPHASE = _RPW // _PHASE
_PRED_PAD = 128
_L = 32768
_TGRID = (_NC + _L - 1) // _L


def _tc_transpose(tt):
    def body(x_ref, o_ref):
        o_ref[...] = jnp.transpose(x_ref[...])

    return pl.pallas_call(
        body,
        grid=(_TGRID,),
        in_specs=[pl.BlockSpec((_CD, _L), lambda i: (0, i))],
        out_specs=pl.BlockSpec((_L, _CD), lambda i: (i, 0)),
        out_shape=jax.ShapeDtypeStruct((_NC, _CD), jnp.float32),
    )(tt)


def _sc_gather(table, idx2):
    mesh = plsc.VectorSubcoreMesh(core_axis_name="c", subcore_axis_name="s")

    @functools.partial(
        pl.kernel,
        mesh=mesh,
        out_type=jax.ShapeDtypeStruct((_ROWS, _WIDE), jnp.float32),
        scratch_types=[
            pltpu.VMEM((_RPW,), jnp.int32),
            pltpu.VMEM((_PHASE, _WIDE), jnp.float32),
            pltpu.SemaphoreType.DMA,
        ],
        compiler_params=pltpu.CompilerParams(needs_layout_passes=False),
    )
    def k(table_hbm, idx_hbm, out_hbm, idx_v, rows_v, sem):
        wid = lax.axis_index("s") * 2 + lax.axis_index("c")
        pltpu.sync_copy(idx_hbm.at[wid], idx_v)
        lanes = lax.broadcasted_iota(jnp.int32, (16,), 0)
        zeros = jnp.zeros((16,), jnp.int32)
        ngroup = _PHASE // 16
        lag = 4

        def drain(i, _):
            pltpu.make_async_copy(
                table_hbm.at[0], rows_v.at[0, pl.ds(0, _CD)], sem
            ).wait()
            return 0

        for ph in range(_NPHASE):
            def group(g, _, base=ph * _PHASE):
                v = idx_v[pl.ds(base + g * 16, 16)]
                for l in range(16):
                    r = jnp.sum(jnp.where(lanes == l, v, zeros))
                    pltpu.async_copy(
                        table_hbm.at[r],
                        rows_v.at[g * 16 + l, pl.ds(0, _CD)],
                        sem,
                    )

                @pl.when(g >= lag)
                def _():
                    lax.fori_loop(0, 16, drain, 0)

                return 0

            lax.fori_loop(0, ngroup, group, 0)
            lax.fori_loop(0, lag * 16, drain, 0)
            pltpu.sync_copy(
                rows_v, out_hbm.at[pl.ds(wid * _RPW + ph * _PHASE, _PHASE)]
            )

    return k(table, idx2)


def _tc_score(g0, g1, pred_idx2, pred_pad, w_p, w0, w1, bias):
    bb = 2048
    grid = _B // bb

    def body(g0_ref, g1_ref, pi_ref, pt_ref, wp_ref, w0_ref, w1_ref, b_ref,
             o_ref):
        p = jnp.dot(pt_ref[...], wp_ref[...], preferred_element_type=jnp.float32)
        onehot = (
            pi_ref[...] == lax.broadcasted_iota(jnp.int32, (bb, _PRED_PAD), 1)
        ).astype(jnp.float32)
        acc = (
            jnp.dot(g0_ref[:, :_CD], w0_ref[...], preferred_element_type=jnp.float32)
            + jnp.dot(g1_ref[:, :_CD], w1_ref[...], preferred_element_type=jnp.float32)
            + jnp.dot(onehot, p, preferred_element_type=jnp.float32)
            + b_ref[...]
        )
        o_ref[...] = jnp.tanh(acc)

    return pl.pallas_call(
        body,
        grid=(grid,),
        in_specs=[
            pl.BlockSpec((bb, _WIDE), lambda i: (i, 0)),
            pl.BlockSpec((bb, _WIDE), lambda i: (i, 0)),
            pl.BlockSpec((bb, 1), lambda i: (i, 0)),
            pl.BlockSpec((_PRED_PAD, _CD), lambda i: (0, 0)),
            pl.BlockSpec((_CD, _CD), lambda i: (0, 0)),
            pl.BlockSpec((_CD, _CD), lambda i: (0, 0)),
            pl.BlockSpec((_CD, _CD), lambda i: (0, 0)),
            pl.BlockSpec((1, _CD), lambda i: (0, 0)),
        ],
        out_specs=pl.BlockSpec((bb, _CD), lambda i: (i, 0)),
        out_shape=jax.ShapeDtypeStruct((_B, _CD), jnp.float32),
    )(g0, g1, pred_idx2, pred_pad, w_p, w0, w1, bias)


def kernel(triplet_idx, predicate_idx, constant_table, predicate_table, W, b):
    ti = triplet_idx.astype(jnp.int32)
    idx_all = jnp.concatenate([ti[:, 0], ti[:, 1]])
    idx2 = idx_all.reshape(_NW, _RPW)
    table_rm = _tc_transpose(constant_table.T)
    g = _sc_gather(table_rm, idx2)
    g0 = g[:_B]
    g1 = g[_B:]
    pred_pad = jnp.zeros((_PRED_PAD, _CD), jnp.float32).at[
        : predicate_table.shape[0]
    ].set(predicate_table)
    pi2 = predicate_idx.astype(jnp.int32).reshape(_B, 1)
    return _tc_score(
        g0, g1, pi2, pred_pad,
        W[:_CD], W[_CD : 2 * _CD], W[2 * _CD :], b.reshape(1, _CD),
    )


# speedup vs baseline: 3.6297x; 1.0347x over previous
import functools

import jax
import jax.numpy as jnp
from jax import lax
from jax.experimental import pallas as pl
from jax.experimental.pallas import tpu as pltpu
from jax.experimental.pallas import tpu_sc as plsc

_B = 16384
_CD = 64
_WIDE = 2 * _CD
_NC = 1000000
_NW = 32
_ROWS = 2 * _B
_RPW = _ROWS // _NW
_PHASE = 512
_NPHASE = _RPW // _PHASE
_PRED_PAD = 128
_L = 32768
_TGRID = (_NC + _L - 1) // _L


def _tc_transpose(tt):
    def body(x_ref, o_ref):
        o_ref[...] = jnp.transpose(x_ref[...])

    return pl.pallas_call(
        body,
        grid=(_TGRID,),
        in_specs=[pl.BlockSpec((_CD, _L), lambda i: (0, i))],
        out_specs=pl.BlockSpec((_L, _CD), lambda i: (i, 0)),
        out_shape=jax.ShapeDtypeStruct((_NC, _CD), jnp.float32),
    )(tt)


def _sc_gather(table, idx2):
    mesh = plsc.VectorSubcoreMesh(core_axis_name="c", subcore_axis_name="s")

    @functools.partial(
        pl.kernel,
        mesh=mesh,
        out_type=jax.ShapeDtypeStruct((_ROWS, _WIDE), jnp.float32),
        scratch_types=[
            pltpu.VMEM((_RPW,), jnp.int32),
            pltpu.VMEM((_PHASE, _WIDE), jnp.float32),
            pltpu.SemaphoreType.DMA,
        ],
        compiler_params=pltpu.CompilerParams(needs_layout_passes=False),
    )
    def k(table_hbm, idx_hbm, out_hbm, idx_v, rows_v, sem):
        wid = lax.axis_index("s") * 2 + lax.axis_index("c")
        pltpu.sync_copy(idx_hbm.at[wid], idx_v)
        lanes = lax.broadcasted_iota(jnp.int32, (16,), 0)
        zeros = jnp.zeros((16,), jnp.int32)
        ngroup = _PHASE // 16
        lag = 4

        def drain(i, _):
            pltpu.make_async_copy(
                table_hbm.at[0], rows_v.at[0, pl.ds(0, _CD)], sem
            ).wait()
            return 0

        for ph in range(_NPHASE):
            def group(g, _, base=ph * _PHASE):
                v = idx_v[pl.ds(base + g * 16, 16)]
                for l in range(16):
                    r = jnp.sum(jnp.where(lanes == l, v, zeros))
                    pltpu.async_copy(
                        table_hbm.at[r],
                        rows_v.at[g * 16 + l, pl.ds(0, _CD)],
                        sem,
                    )

                @pl.when(g >= lag)
                def _():
                    lax.fori_loop(0, 16, drain, 0)

                return 0

            lax.fori_loop(0, ngroup, group, 0)
            lax.fori_loop(0, lag * 16, drain, 0)
            pltpu.sync_copy(
                rows_v, out_hbm.at[pl.ds(wid * _RPW + ph * _PHASE, _PHASE)]
            )

    return k(table, idx2)


def _tc_score(g0, g1, pred_idx2, pred_pad, w_p, w0, w1, bias):
    bb = 2048
    grid = _B // bb

    def body(g0_ref, g1_ref, pi_ref, pt_ref, wp_ref, w0_ref, w1_ref, b_ref,
             o_ref):
        p = jnp.dot(pt_ref[...], wp_ref[...], preferred_element_type=jnp.float32)
        onehot = (
            pi_ref[...] == lax.broadcasted_iota(jnp.int32, (bb, _PRED_PAD), 1)
        ).astype(jnp.float32)
        acc = (
            jnp.dot(g0_ref[:, :_CD], w0_ref[...], preferred_element_type=jnp.float32)
            + jnp.dot(g1_ref[:, :_CD], w1_ref[...], preferred_element_type=jnp.float32)
            + jnp.dot(onehot, p, preferred_element_type=jnp.float32)
            + b_ref[...]
        )
        o_ref[...] = jnp.tanh(acc)

    return pl.pallas_call(
        body,
        grid=(grid,),
        in_specs=[
            pl.BlockSpec((bb, _WIDE), lambda i: (i, 0)),
            pl.BlockSpec((bb, _WIDE), lambda i: (i + _B // bb, 0)),
            pl.BlockSpec((bb, 1), lambda i: (i, 0)),
            pl.BlockSpec((_PRED_PAD, _CD), lambda i: (0, 0)),
            pl.BlockSpec((_CD, _CD), lambda i: (0, 0)),
            pl.BlockSpec((_CD, _CD), lambda i: (0, 0)),
            pl.BlockSpec((_CD, _CD), lambda i: (0, 0)),
            pl.BlockSpec((1, _CD), lambda i: (0, 0)),
        ],
        out_specs=pl.BlockSpec((bb, _CD), lambda i: (i, 0)),
        out_shape=jax.ShapeDtypeStruct((_B, _CD), jnp.float32),
    )(g0, g1, pred_idx2, pred_pad, w_p, w0, w1, bias)


def kernel(triplet_idx, predicate_idx, constant_table, predicate_table, W, b):
    ti = triplet_idx.astype(jnp.int32)
    idx_all = jnp.concatenate([ti[:, 0], ti[:, 1]])
    idx2 = idx_all.reshape(_NW, _RPW)
    table_rm = _tc_transpose(constant_table.T)
    g = _sc_gather(table_rm, idx2)
    pred_pad = jnp.zeros((_PRED_PAD, _CD), jnp.float32).at[
        : predicate_table.shape[0]
    ].set(predicate_table)
    pi2 = predicate_idx.astype(jnp.int32).reshape(_B, 1)
    return _tc_score(
        g, g, pi2, pred_pad,
        W[:_CD], W[_CD : 2 * _CD], W[2 * _CD :], b.reshape(1, _CD),
    )
